# baseline (device time: 224783 ns/iter reference)
import jax
import jax.numpy as jnp
from jax import lax
from jax.experimental import pallas as pl
from jax.experimental.pallas import tpu as pltpu

N_DEV = 16
B_LOC = 2
SQ = 128
SKV = 128
H_LOC = 4
DH = 64
D_MODEL = 512
HD = H_LOC * DH


def _body(x_ref, wq_ref, wo_ref, k_ref, v_ref, out_ref,
          wq_g, wo_g, send_wq, recv_wq, send_wo, recv_wo):
    my = lax.axis_index("i")
    right = lax.rem(my + 1, N_DEV)
    left = lax.rem(my + N_DEV - 1, N_DEV)

    barrier = pltpu.get_barrier_semaphore()
    pl.semaphore_signal(barrier, inc=1, device_id=(left,),
                        device_id_type=pl.DeviceIdType.MESH)
    pl.semaphore_signal(barrier, inc=1, device_id=(right,),
                        device_id_type=pl.DeviceIdType.MESH)
    pl.semaphore_wait(barrier, 2)

    x2d = x_ref[...].reshape(B_LOC * SQ, D_MODEL)

    row_blk = lax.broadcasted_iota(jnp.int32, (SQ, SKV), 0) // 64
    col_blk = lax.broadcasted_iota(jnp.int32, (SQ, SKV), 1) // 64
    mask = (row_blk == col_blk) | (
        lax.rem(col_blk, 4) == lax.rem(row_blk, 4))

    def compute_chunk(c, wq_c, wo_c, init):
        q = jnp.dot(x2d, wq_c, preferred_element_type=jnp.float32)
        for b in range(B_LOC):
            k_blk = k_ref[b, pl.ds(c * H_LOC, H_LOC), :, :]
            v_blk = v_ref[b, pl.ds(c * H_LOC, H_LOC), :, :]
            acc = jnp.zeros((SQ, D_MODEL), jnp.float32)
            for h in range(H_LOC):
                q_bh = q[b * SQ:(b + 1) * SQ, h * DH:(h + 1) * DH]
                s = lax.dot_general(
                    q_bh, k_blk[h], (((1,), (1,)), ((), ())),
                    preferred_element_type=jnp.float32) * 0.125
                s = jnp.where(mask, s, -1e9)
                m = jnp.max(s, axis=-1, keepdims=True)
                e = jnp.exp(s - m)
                w = e / jnp.sum(e, axis=-1, keepdims=True)
                ctx = jnp.dot(w, v_blk[h],
                              preferred_element_type=jnp.float32)
                acc = acc + jnp.dot(ctx, wo_c[h * DH:(h + 1) * DH, :],
                                    preferred_element_type=jnp.float32)
            if init:
                out_ref[b, :, :] = acc
            else:
                out_ref[b, :, :] += acc

    for h in range(N_DEV - 1):
        slot = lax.rem(my - h + N_DEV, N_DEV)
        rdma_wq = pltpu.make_async_remote_copy(
            src_ref=wq_ref if h == 0 else wq_g.at[slot],
            dst_ref=wq_g.at[slot],
            send_sem=send_wq.at[h], recv_sem=recv_wq.at[h],
            device_id=(right,), device_id_type=pl.DeviceIdType.MESH)
        rdma_wo = pltpu.make_async_remote_copy(
            src_ref=wo_ref if h == 0 else wo_g.at[slot],
            dst_ref=wo_g.at[slot],
            send_sem=send_wo.at[h], recv_sem=recv_wo.at[h],
            device_id=(right,), device_id_type=pl.DeviceIdType.MESH)
        rdma_wq.start()
        rdma_wo.start()
        if h == 0:
            compute_chunk(my, wq_ref[...], wo_ref[...], init=True)
        else:
            compute_chunk(
                slot,
                wq_g[pl.ds(slot, 1), :, :].reshape(D_MODEL, HD),
                wo_g[pl.ds(slot, 1), :, :].reshape(HD, D_MODEL),
                init=False)
        rdma_wq.wait()
        rdma_wo.wait()

    last = lax.rem(my + 1, N_DEV)
    compute_chunk(
        last,
        wq_g[pl.ds(last, 1), :, :].reshape(D_MODEL, HD),
        wo_g[pl.ds(last, 1), :, :].reshape(HD, D_MODEL),
        init=False)


def kernel(x, Wq, K_ext, V_ext, Wo):
    my = lax.axis_index("i")
    k_my = lax.dynamic_slice_in_dim(K_ext, my * B_LOC, B_LOC, axis=0)
    v_my = lax.dynamic_slice_in_dim(V_ext, my * B_LOC, B_LOC, axis=0)
    k_t = jnp.transpose(k_my, (0, 2, 1, 3))
    v_t = jnp.transpose(v_my, (0, 2, 1, 3))

    return pl.pallas_call(
        _body,
        out_shape=jax.ShapeDtypeStruct((B_LOC, SQ, D_MODEL), jnp.float32),
        in_specs=[pl.BlockSpec(memory_space=pltpu.VMEM)] * 5,
        out_specs=pl.BlockSpec(memory_space=pltpu.VMEM),
        scratch_shapes=[
            pltpu.VMEM((N_DEV, D_MODEL, HD), jnp.float32),
            pltpu.VMEM((N_DEV, HD, D_MODEL), jnp.float32),
            pltpu.SemaphoreType.DMA((N_DEV - 1,)),
            pltpu.SemaphoreType.DMA((N_DEV - 1,)),
            pltpu.SemaphoreType.DMA((N_DEV - 1,)),
            pltpu.SemaphoreType.DMA((N_DEV - 1,)),
        ],
        compiler_params=pltpu.CompilerParams(
            collective_id=0, vmem_limit_bytes=100 * 1024 * 1024),
    )(x, Wq, Wo, k_t, v_t)


# device time: 101999 ns/iter; 2.2038x vs baseline; 2.2038x over previous
import jax
import jax.numpy as jnp
from jax import lax
from jax.experimental import pallas as pl
from jax.experimental.pallas import tpu as pltpu

N_DEV = 16
B_LOC = 2
SQ = 128
SKV = 128
H_LOC = 4
DH = 64
D_MODEL = 512
HD = H_LOC * DH

R_HOPS = N_DEV // 2
L_HOPS = N_DEV - 1 - R_HOPS


def _body(x_ref, wq_ref, wo_ref, k_ref, v_ref, out_ref,
          wq_g, wo_g, send_sems, recv_sems):
    my = lax.axis_index("i")
    right = lax.rem(my + 1, N_DEV)
    left = lax.rem(my + N_DEV - 1, N_DEV)

    barrier = pltpu.get_barrier_semaphore()
    pl.semaphore_signal(barrier, inc=1, device_id=(left,),
                        device_id_type=pl.DeviceIdType.MESH)
    pl.semaphore_signal(barrier, inc=1, device_id=(right,),
                        device_id_type=pl.DeviceIdType.MESH)
    pl.semaphore_wait(barrier, 2)

    x2d = x_ref[...].reshape(B_LOC * SQ, D_MODEL)

    row_blk = lax.broadcasted_iota(jnp.int32, (SQ, SKV), 0) // 64
    col_blk = lax.broadcasted_iota(jnp.int32, (SQ, SKV), 1) // 64
    mask = (row_blk == col_blk) | (
        lax.rem(col_blk, 4) == lax.rem(row_blk, 4))

    def compute_chunk(c, init=False):
        wq_c = wq_g[pl.ds(c, 1), :, :].reshape(D_MODEL, HD).astype(jnp.float32)
        wo_c = wo_g[pl.ds(c, 1), :, :].reshape(HD, D_MODEL).astype(jnp.float32)
        q = jnp.dot(x2d, wq_c, preferred_element_type=jnp.float32)
        for b in range(B_LOC):
            k_blk = k_ref[b, pl.ds(c * H_LOC, H_LOC), :, :]
            v_blk = v_ref[b, pl.ds(c * H_LOC, H_LOC), :, :]
            acc = jnp.zeros((SQ, D_MODEL), jnp.float32)
            for h in range(H_LOC):
                q_bh = q[b * SQ:(b + 1) * SQ, h * DH:(h + 1) * DH]
                s = lax.dot_general(
                    q_bh, k_blk[h], (((1,), (1,)), ((), ())),
                    preferred_element_type=jnp.float32) * 0.125
                s = jnp.where(mask, s, -1e9)
                m = jnp.max(s, axis=-1, keepdims=True)
                e = jnp.exp(s - m)
                w = e / jnp.sum(e, axis=-1, keepdims=True)
                ctx = jnp.dot(w, v_blk[h],
                              preferred_element_type=jnp.float32)
                acc = acc + jnp.dot(ctx, wo_c[h * DH:(h + 1) * DH, :],
                                    preferred_element_type=jnp.float32)
            if init:
                out_ref[b, :, :] = acc
            else:
                out_ref[b, :, :] += acc

    wq_g[pl.ds(my, 1), :, :] = wq_ref[...].reshape(1, D_MODEL, HD)
    wo_g[pl.ds(my, 1), :, :] = wo_ref[...].reshape(1, HD, D_MODEL)

    def make_rdma(buf, slot, sem_row, h, dev):
        return pltpu.make_async_remote_copy(
            src_ref=buf.at[slot], dst_ref=buf.at[slot],
            send_sem=send_sems.at[sem_row, h],
            recv_sem=recv_sems.at[sem_row, h],
            device_id=(dev,), device_id_type=pl.DeviceIdType.MESH)

    for h in range(R_HOPS + 1):
        rdmas = []
        if h < R_HOPS:
            slot_r = lax.rem(my - h + N_DEV, N_DEV)
            rdmas.append(make_rdma(wq_g, slot_r, 0, h, right))
            rdmas.append(make_rdma(wo_g, slot_r, 1, h, right))
        if h < L_HOPS:
            slot_l = lax.rem(my + h, N_DEV)
            rdmas.append(make_rdma(wq_g, slot_l, 2, h, left))
            rdmas.append(make_rdma(wo_g, slot_l, 3, h, left))
        for r in rdmas:
            r.start()
        if h == 0:
            compute_chunk(my, init=True)
        else:
            compute_chunk(lax.rem(my - h + N_DEV, N_DEV))
            if h <= L_HOPS:
                compute_chunk(lax.rem(my + h, N_DEV))
        for r in rdmas:
            r.wait()


def kernel(x, Wq, K_ext, V_ext, Wo):
    my = lax.axis_index("i")
    k_my = lax.dynamic_slice_in_dim(K_ext, my * B_LOC, B_LOC, axis=0)
    v_my = lax.dynamic_slice_in_dim(V_ext, my * B_LOC, B_LOC, axis=0)
    k_t = jnp.transpose(k_my, (0, 2, 1, 3))
    v_t = jnp.transpose(v_my, (0, 2, 1, 3))
    wq_b = Wq.astype(jnp.bfloat16)
    wo_b = Wo.astype(jnp.bfloat16)

    return pl.pallas_call(
        _body,
        out_shape=jax.ShapeDtypeStruct((B_LOC, SQ, D_MODEL), jnp.float32),
        in_specs=[pl.BlockSpec(memory_space=pltpu.VMEM)] * 5,
        out_specs=pl.BlockSpec(memory_space=pltpu.VMEM),
        scratch_shapes=[
            pltpu.VMEM((N_DEV, D_MODEL, HD), jnp.bfloat16),
            pltpu.VMEM((N_DEV, HD, D_MODEL), jnp.bfloat16),
            pltpu.SemaphoreType.DMA((4, R_HOPS)),
            pltpu.SemaphoreType.DMA((4, R_HOPS)),
        ],
        compiler_params=pltpu.CompilerParams(
            collective_id=0, vmem_limit_bytes=100 * 1024 * 1024),
    )(x, wq_b, wo_b, k_t, v_t)


# device time: 97840 ns/iter; 2.2975x vs baseline; 1.0425x over previous
import jax
import jax.numpy as jnp
from jax import lax
from jax.experimental import pallas as pl
from jax.experimental.pallas import tpu as pltpu

N_DEV = 16
B_LOC = 2
SQ = 128
SKV = 128
H_LOC = 4
DH = 64
D_MODEL = 512
HD = H_LOC * DH

R_HOPS = N_DEV // 2
L_HOPS = N_DEV - 1 - R_HOPS


def _body(x_ref, wq_ref, wo_ref, k_ref, v_ref, out_ref,
          wq_g, wo_g, send_sems, recv_sems):
    my = lax.axis_index("i")
    right = lax.rem(my + 1, N_DEV)
    left = lax.rem(my + N_DEV - 1, N_DEV)

    barrier = pltpu.get_barrier_semaphore()
    pl.semaphore_signal(barrier, inc=1, device_id=(left,),
                        device_id_type=pl.DeviceIdType.MESH)
    pl.semaphore_signal(barrier, inc=1, device_id=(right,),
                        device_id_type=pl.DeviceIdType.MESH)
    pl.semaphore_wait(barrier, 2)

    x2d = x_ref[...].reshape(B_LOC * SQ, D_MODEL)

    row_blk = lax.broadcasted_iota(jnp.int32, (SQ, SKV), 0) // 64
    col_blk = lax.broadcasted_iota(jnp.int32, (SQ, SKV), 1) // 64
    mask = (row_blk == col_blk) | (
        lax.rem(col_blk, 4) == lax.rem(row_blk, 4))

    def compute_chunk(c, init=False):
        wq_c = wq_g[pl.ds(c, 1), :, :].reshape(D_MODEL, HD)
        wo_c = wo_g[pl.ds(c, 1), :, :].reshape(HD, D_MODEL)
        q = jnp.dot(x2d, wq_c, preferred_element_type=jnp.float32
                    ).astype(jnp.bfloat16)
        for b in range(B_LOC):
            k_blk = k_ref[b, pl.ds(c * H_LOC, H_LOC), :, :]
            v_blk = v_ref[b, pl.ds(c * H_LOC, H_LOC), :, :]
            acc = jnp.zeros((SQ, D_MODEL), jnp.float32)
            for h in range(H_LOC):
                q_bh = q[b * SQ:(b + 1) * SQ, h * DH:(h + 1) * DH]
                s = lax.dot_general(
                    q_bh, k_blk[h], (((1,), (1,)), ((), ())),
                    preferred_element_type=jnp.float32) * 0.125
                s = jnp.where(mask, s, -1e9)
                m = jnp.max(s, axis=-1, keepdims=True)
                e = jnp.exp(s - m)
                w = (e / jnp.sum(e, axis=-1, keepdims=True)
                     ).astype(jnp.bfloat16)
                ctx = jnp.dot(w, v_blk[h], preferred_element_type=jnp.float32
                              ).astype(jnp.bfloat16)
                acc = acc + jnp.dot(ctx, wo_c[h * DH:(h + 1) * DH, :],
                                    preferred_element_type=jnp.float32)
            if init:
                out_ref[b, :, :] = acc
            else:
                out_ref[b, :, :] += acc

    wq_g[pl.ds(my, 1), :, :] = wq_ref[...].reshape(1, D_MODEL, HD)
    wo_g[pl.ds(my, 1), :, :] = wo_ref[...].reshape(1, HD, D_MODEL)

    def make_rdma(buf, slot, sem_row, h, dev):
        return pltpu.make_async_remote_copy(
            src_ref=buf.at[slot], dst_ref=buf.at[slot],
            send_sem=send_sems.at[sem_row, h],
            recv_sem=recv_sems.at[sem_row, h],
            device_id=(dev,), device_id_type=pl.DeviceIdType.MESH)

    for h in range(R_HOPS + 1):
        rdmas = []
        if h < R_HOPS:
            slot_r = lax.rem(my - h + N_DEV, N_DEV)
            rdmas.append(make_rdma(wq_g, slot_r, 0, h, right))
            rdmas.append(make_rdma(wo_g, slot_r, 1, h, right))
        if h < L_HOPS:
            slot_l = lax.rem(my + h, N_DEV)
            rdmas.append(make_rdma(wq_g, slot_l, 2, h, left))
            rdmas.append(make_rdma(wo_g, slot_l, 3, h, left))
        for r in rdmas:
            r.start()
        if h == 0:
            compute_chunk(my, init=True)
        else:
            compute_chunk(lax.rem(my - h + N_DEV, N_DEV))
            if h <= L_HOPS:
                compute_chunk(lax.rem(my + h, N_DEV))
        for r in rdmas:
            r.wait()


def kernel(x, Wq, K_ext, V_ext, Wo):
    my = lax.axis_index("i")
    k_my = lax.dynamic_slice_in_dim(K_ext, my * B_LOC, B_LOC, axis=0)
    v_my = lax.dynamic_slice_in_dim(V_ext, my * B_LOC, B_LOC, axis=0)
    k_t = jnp.transpose(k_my, (0, 2, 1, 3)).astype(jnp.bfloat16)
    v_t = jnp.transpose(v_my, (0, 2, 1, 3)).astype(jnp.bfloat16)
    wq_b = Wq.astype(jnp.bfloat16)
    wo_b = Wo.astype(jnp.bfloat16)
    x_b = x.astype(jnp.bfloat16)

    return pl.pallas_call(
        _body,
        out_shape=jax.ShapeDtypeStruct((B_LOC, SQ, D_MODEL), jnp.float32),
        in_specs=[pl.BlockSpec(memory_space=pltpu.VMEM)] * 5,
        out_specs=pl.BlockSpec(memory_space=pltpu.VMEM),
        scratch_shapes=[
            pltpu.VMEM((N_DEV, D_MODEL, HD), jnp.bfloat16),
            pltpu.VMEM((N_DEV, HD, D_MODEL), jnp.bfloat16),
            pltpu.SemaphoreType.DMA((4, R_HOPS)),
            pltpu.SemaphoreType.DMA((4, R_HOPS)),
        ],
        compiler_params=pltpu.CompilerParams(
            collective_id=0, vmem_limit_bytes=100 * 1024 * 1024),
    )(x_b, wq_b, wo_b, k_t, v_t)


# device time: 84020 ns/iter; 2.6754x vs baseline; 1.1645x over previous
import jax
import jax.numpy as jnp
from jax import lax
from jax.experimental import pallas as pl
from jax.experimental.pallas import tpu as pltpu

N_DEV = 16
B_LOC = 2
SQ = 128
SKV = 128
H_LOC = 4
DH = 64
D_MODEL = 512
HD = H_LOC * DH

R_HOPS = N_DEV // 2
L_HOPS = N_DEV - 1 - R_HOPS

RING = [0, 4, 8, 12, 15, 11, 7, 3, 2, 6, 10, 14, 13, 9, 5, 1]
POS = [0] * N_DEV
for _p, _l in enumerate(RING):
    POS[_l] = _p


def _lut(table, idx):
    out = jnp.int32(table[0])
    for i in range(1, N_DEV):
        out = jnp.where(idx == i, jnp.int32(table[i]), out)
    return out


def _body(x_ref, w_ref, k_ref, v_ref, out_ref, w_g, send_sems, recv_sems):
    my = lax.axis_index("i")
    r = _lut(POS, my)
    right = _lut(RING, lax.rem(r + 1, N_DEV))
    left = _lut(RING, lax.rem(r + N_DEV - 1, N_DEV))

    barrier = pltpu.get_barrier_semaphore()
    pl.semaphore_signal(barrier, inc=1, device_id=(left,),
                        device_id_type=pl.DeviceIdType.MESH)
    pl.semaphore_signal(barrier, inc=1, device_id=(right,),
                        device_id_type=pl.DeviceIdType.MESH)
    pl.semaphore_wait(barrier, 2)

    xT = x_ref[...]

    row_blk = lax.broadcasted_iota(jnp.int32, (SQ, SKV), 0) // 64
    col_blk = lax.broadcasted_iota(jnp.int32, (SQ, SKV), 1) // 64
    mask = (row_blk == col_blk) | (
        lax.rem(col_blk, 4) == lax.rem(row_blk, 4))

    def compute_chunk(c, init=False):
        wqt_c = w_g[pl.ds(c, 1), 0, :, :].reshape(HD, D_MODEL)
        wo_c = w_g[pl.ds(c, 1), 1, :, :].reshape(HD, D_MODEL)
        wo4 = wo_c.reshape(H_LOC, DH, D_MODEL)
        qt = lax.dot_general(wqt_c, xT, (((1,), (0,)), ((), ())),
                             preferred_element_type=jnp.float32
                             ).astype(jnp.bfloat16)
        qt4 = qt.reshape(H_LOC, DH, B_LOC * SQ)
        for b in range(B_LOC):
            kb = k_ref[b, pl.ds(c * H_LOC, H_LOC), :, :]
            vb = v_ref[b, pl.ds(c * H_LOC, H_LOC), :, :]
            qtb = qt4[:, :, b * SQ:(b + 1) * SQ]
            st = lax.dot_general(
                kb, qtb, (((2,), (1,)), ((0,), (0,))),
                preferred_element_type=jnp.float32)
            e = jnp.exp(jnp.where(mask[None], st, -1e9))
            w = (e / jnp.sum(e, axis=1, keepdims=True)).astype(jnp.bfloat16)
            ctx = lax.dot_general(
                w, vb, (((1,), (1,)), ((0,), (0,))),
                preferred_element_type=jnp.float32
                ).astype(jnp.bfloat16)
            part = lax.dot_general(
                ctx, wo4, (((2,), (1,)), ((0,), (0,))),
                preferred_element_type=jnp.float32)
            contrib = jnp.sum(part, axis=0)
            if init:
                out_ref[b, :, :] = contrib
            else:
                out_ref[b, :, :] += contrib

    w_g[pl.ds(my, 1), :, :, :] = w_ref[...].reshape(1, 2, HD, D_MODEL)

    def make_rdma(slot, sem_row, h, dev):
        return pltpu.make_async_remote_copy(
            src_ref=w_g.at[slot], dst_ref=w_g.at[slot],
            send_sem=send_sems.at[sem_row, h],
            recv_sem=recv_sems.at[sem_row, h],
            device_id=(dev,), device_id_type=pl.DeviceIdType.MESH)

    for h in range(R_HOPS + 1):
        rdmas = []
        if h < R_HOPS:
            rdmas.append(make_rdma(_lut(RING, lax.rem(r - h + N_DEV, N_DEV)),
                                   0, h, right))
        if h < L_HOPS:
            rdmas.append(make_rdma(_lut(RING, lax.rem(r + h, N_DEV)),
                                   1, h, left))
        for rd in rdmas:
            rd.start()
        if h == 0:
            compute_chunk(my, init=True)
        else:
            compute_chunk(_lut(RING, lax.rem(r - h + N_DEV, N_DEV)))
            if h <= L_HOPS:
                compute_chunk(_lut(RING, lax.rem(r + h, N_DEV)))
        for rd in rdmas:
            rd.wait()


def kernel(x, Wq, K_ext, V_ext, Wo):
    my = lax.axis_index("i")
    k_my = lax.dynamic_slice_in_dim(K_ext, my * B_LOC, B_LOC, axis=0)
    v_my = lax.dynamic_slice_in_dim(V_ext, my * B_LOC, B_LOC, axis=0)
    k_t = jnp.transpose(k_my, (0, 2, 1, 3)).astype(jnp.bfloat16)
    v_t = jnp.transpose(v_my, (0, 2, 1, 3)).astype(jnp.bfloat16)
    w_pay = jnp.stack([Wq.T, Wo]).astype(jnp.bfloat16)
    x_t = (x.reshape(B_LOC * SQ, D_MODEL).T * 0.125).astype(jnp.bfloat16)

    return pl.pallas_call(
        _body,
        out_shape=jax.ShapeDtypeStruct((B_LOC, SQ, D_MODEL), jnp.float32),
        in_specs=[pl.BlockSpec(memory_space=pltpu.VMEM)] * 4,
        out_specs=pl.BlockSpec(memory_space=pltpu.VMEM),
        scratch_shapes=[
            pltpu.VMEM((N_DEV, 2, HD, D_MODEL), jnp.bfloat16),
            pltpu.SemaphoreType.DMA((2, R_HOPS)),
            pltpu.SemaphoreType.DMA((2, R_HOPS)),
        ],
        compiler_params=pltpu.CompilerParams(
            collective_id=0, vmem_limit_bytes=100 * 1024 * 1024),
    )(x_t, w_pay, k_t, v_t)
